# Initial kernel scaffold; baseline (speedup 1.0000x reference)
import jax
import jax.numpy as jnp
from jax.experimental import pallas as pl


def kernel(x, lgx, edge_index, W_q, b_q, W_k, W_v, W_o, b_o, ln1_g, ln1_b, W1, b1, W2, b2, ln2_g, ln2_b):
    raise NotImplementedError("write your pallas kernel here")


# speedup vs baseline: 17.3288x; 17.3288x over previous
import functools

import jax
import jax.numpy as jnp
from jax import lax
from jax.experimental import pallas as pl
from jax.experimental.pallas import tpu as pltpu
from jax.experimental.pallas import tpu_sc as plsc

N = 10000
E = 320000
NDIM = 128
EDIM = 16
H = 8
DK = 16

NC = 2
NS = 16
NW = NC * NS
EPW = E // NW
C = 40
NCHUNK = EPW // C
RCH = 16
NRCH = N // RCH
NBCH = (NRCH + NS - 1) // NS
NG = N // 8
NGCH = (NG + RCH - 1) // RCH
NGP = NGCH * RCH
NBG = (NGCH + NS - 1) // NS


def _qkv_body(x_ref, w_ref, b_ref, q_ref, k_ref, v_ref):
    acc = jnp.dot(x_ref[...], w_ref[...], preferred_element_type=jnp.float32)
    acc = acc + b_ref[...]
    q_ref[...] = acc[:, 0:NDIM]
    k_ref[...] = acc[:, NDIM:2 * NDIM]
    v_ref[...] = acc[:, 2 * NDIM:3 * NDIM]


def _qkv_call(x, wqkv, bqkv):
    br = 1000
    grid = (N // br,)
    return pl.pallas_call(
        _qkv_body,
        grid=grid,
        in_specs=[
            pl.BlockSpec((br, NDIM), lambda i: (i, 0)),
            pl.BlockSpec((NDIM, 3 * NDIM), lambda i: (0, 0)),
            pl.BlockSpec((1, 3 * NDIM), lambda i: (0, 0)),
        ],
        out_specs=[
            pl.BlockSpec((br, NDIM), lambda i: (i, 0)),
            pl.BlockSpec((br, NDIM), lambda i: (i, 0)),
            pl.BlockSpec((br, NDIM), lambda i: (i, 0)),
        ],
        out_shape=[
            jax.ShapeDtypeStruct((N, NDIM), jnp.float32),
            jax.ShapeDtypeStruct((N, NDIM), jnp.float32),
            jax.ShapeDtypeStruct((N, NDIM), jnp.float32),
        ],
    )(x, wqkv, bqkv)


def _dyng(v, idx):
    return lax.gather(
        v, idx[:, None],
        dimension_numbers=lax.GatherDimensionNumbers(
            offset_dims=(), collapsed_slice_dims=(0,), start_index_map=(0,)),
        slice_sizes=(1,), mode=lax.GatherScatterMode.PROMISE_IN_BOUNDS)


_SC_MESH = plsc.VectorSubcoreMesh(core_axis_name="c", subcore_axis_name="s")


@functools.partial(
    pl.kernel,
    mesh=_SC_MESH,
    out_type=(
        jax.ShapeDtypeStruct((NC, N, NDIM), jnp.float32),
        jax.ShapeDtypeStruct((NC, NGP, NDIM), jnp.float32),
    ),
    scratch_types=[
        pltpu.VMEM((C,), jnp.int32),
        pltpu.VMEM((C + 16,), jnp.int32),
        pltpu.VMEM((C,), jnp.int32),
        pltpu.VMEM((C, NDIM), jnp.float32),
        pltpu.VMEM((C, NDIM), jnp.float32),
        pltpu.VMEM((C, NDIM), jnp.float32),
        pltpu.VMEM((C, EDIM), jnp.float32),
        pltpu.VMEM((C, NDIM), jnp.float32),
        pltpu.VMEM((C, NDIM), jnp.float32),
        pltpu.VMEM((C,), jnp.int32),
        pltpu.VMEM((16,), jnp.int32),
        pltpu.VMEM_SHARED((N, NDIM), jnp.float32),
        pltpu.VMEM_SHARED((NGP, NDIM), jnp.float32),
        pltpu.SemaphoreType.DMA,
        pltpu.SemaphoreType.DMA,
        pltpu.SemaphoreType.DMA,
        pltpu.SemaphoreType.DMA,
    ],
)
def _edge_kernel(q_hbm, k_hbm, v_hbm, lgx_hbm, src_hbm, dst_hbm, dstg_hbm,
                 wv_out, z_out,
                 src_v, dst_v, dsti_v, k_v, q_v, v_v, e_v, msg_v, zp_v, dstg_v, idx_v,
                 wv_sh, z_sh, sem0, sem1, sem2, sem3):
    cid = lax.axis_index("c")
    sid = lax.axis_index("s")
    wid = sid * NC + cid

    zv = jnp.zeros((16,), jnp.float32)

    def _zrow(r, carry):
        for j in range(NDIM // 16):
            msg_v[r, pl.ds(j * 16, 16)] = zv
            zp_v[r, pl.ds(j * 16, 16)] = zv
        return carry
    lax.fori_loop(0, C, _zrow, 0)

    lane = lax.iota(jnp.int32, 16)

    def _zchunk(i, carry):
        off = jnp.minimum(sid + i * NS, NRCH - 1) * RCH
        idx_v[pl.ds(0, 16)] = off + lane
        pltpu.async_copy(msg_v.at[pl.ds(0, RCH)],
                         wv_sh.at[idx_v], sem0).wait()
        return carry
    lax.fori_loop(0, NBCH, _zchunk, 0)

    def _zgchunk(i, carry):
        off = jnp.minimum(sid + i * NS, NGCH - 1) * RCH
        idx_v[pl.ds(0, 16)] = off + lane
        pltpu.async_copy(zp_v.at[pl.ds(0, RCH)],
                         z_sh.at[idx_v], sem1).wait()
        return carry
    lax.fori_loop(0, NBG, _zgchunk, 0)
    plsc.subcore_barrier()

    def _chunk(ci, carry):
        gbase = wid * EPW + ci * C
        pltpu.async_copy(src_hbm.at[pl.ds(gbase, C)], src_v, sem0).wait()
        pltpu.async_copy(dst_hbm.at[pl.ds(gbase, C)],
                         dst_v.at[pl.ds(0, C)], sem1).wait()
        pltpu.async_copy(dst_hbm.at[pl.ds(gbase, C)], dsti_v, sem3).wait()
        pltpu.async_copy(dstg_hbm.at[pl.ds(gbase, C)], dstg_v, sem2).wait()
        pltpu.async_copy(k_hbm.at[src_v], k_v, sem0).wait()
        pltpu.async_copy(v_hbm.at[src_v], v_v, sem1).wait()
        pltpu.async_copy(q_hbm.at[dsti_v], q_v, sem2).wait()
        pltpu.async_copy(lgx_hbm.at[pl.ds(gbase, C)], e_v, sem3).wait()

        def _edge(ei, ecarry):
            ef = e_v[ei, pl.ds(0, EDIM)]
            zvec = jnp.zeros((16,), jnp.float32)
            for h in range(H):
                kk = k_v[ei, pl.ds(h * DK, DK)]
                qq = q_v[ei, pl.ds(h * DK, DK)]
                p = (kk + ef) * qq
                for st in (8, 4, 2, 1):
                    p = p + _dyng(p, lane ^ st)
                zvec = jnp.where(lane == h, p, zvec)
            ez = jnp.exp(jnp.minimum(jnp.maximum(zvec * 0.25, -5.0), 5.0))
            for h in range(H):
                ph = _dyng(ez, jnp.full((16,), h, jnp.int32))
                vv = v_v[ei, pl.ds(h * DK, DK)]
                msg_v[ei, pl.ds(h * DK, DK)] = (vv + ef) * ph
            base = (ei // 16) * 16
            dvec = dst_v[pl.ds(base, 16)] & 7
            slot = _dyng(dvec, jnp.full((16,), ei - base, jnp.int32))
            slotf = slot.astype(jnp.float32)
            for j in range(8):
                d = slotf - float(j)
                ind = jnp.maximum(1.0 - d * d, 0.0)
                zp_v[ei, pl.ds(j * 16, 16)] = ez * ind
            return ecarry
        lax.fori_loop(0, C, _edge, 0)

        pltpu.async_copy(msg_v, wv_sh.at[dsti_v], sem0, add=True).wait()
        pltpu.async_copy(zp_v, z_sh.at[dstg_v], sem1, add=True).wait()
        return carry
    lax.fori_loop(0, NCHUNK, _chunk, 0)

    plsc.subcore_barrier()

    def _wchunk(i, carry):
        off = jnp.minimum(sid + i * NS, NRCH - 1) * RCH
        idx_v[pl.ds(0, 16)] = off + lane
        pltpu.async_copy(wv_sh.at[idx_v],
                         msg_v.at[pl.ds(0, RCH)], sem0).wait()
        pltpu.async_copy(msg_v.at[pl.ds(0, RCH)],
                         wv_out.at[cid, pl.ds(off, RCH)], sem1).wait()
        return carry
    lax.fori_loop(0, NBCH, _wchunk, 0)

    def _wgchunk(i, carry):
        off = jnp.minimum(sid + i * NS, NGCH - 1) * RCH
        idx_v[pl.ds(0, 16)] = off + lane
        pltpu.async_copy(z_sh.at[idx_v],
                         zp_v.at[pl.ds(0, RCH)], sem2).wait()
        pltpu.async_copy(zp_v.at[pl.ds(0, RCH)],
                         z_out.at[cid, pl.ds(off, RCH)], sem3).wait()
        return carry
    lax.fori_loop(0, NBG, _wgchunk, 0)


def _post_body(x_ref, wv_ref, z_ref, wo_ref, bo_ref, g1_ref, b1_ref,
               w1_ref, bb1_ref, w2_ref, bb2_ref, g2_ref, b2_ref, out_ref):
    wv = wv_ref[0] + wv_ref[1]
    z = z_ref[0] + z_ref[1]
    rows = lax.broadcasted_iota(jnp.int32, (16, NDIM), 0)
    cols = lax.broadcasted_iota(jnp.int32, (16, NDIM), 1)
    sel = (rows == cols // DK).astype(jnp.float32)
    zfull = jnp.dot(z, sel, preferred_element_type=jnp.float32)
    o = wv / zfull

    t = x_ref[...] + jnp.dot(o, wo_ref[...],
                             preferred_element_type=jnp.float32) + bo_ref[...]
    mu = jnp.mean(t, axis=-1, keepdims=True)
    var = jnp.mean((t - mu) ** 2, axis=-1, keepdims=True)
    t1 = (t - mu) / jnp.sqrt(var + 1e-5) * g1_ref[...] + b1_ref[...]

    hdn = jnp.maximum(
        jnp.dot(t1, w1_ref[...], preferred_element_type=jnp.float32)
        + bb1_ref[...], 0.0)
    ff = jnp.dot(hdn, w2_ref[...],
                 preferred_element_type=jnp.float32) + bb2_ref[...]

    t2 = t1 + ff
    mu2 = jnp.mean(t2, axis=-1, keepdims=True)
    var2 = jnp.mean((t2 - mu2) ** 2, axis=-1, keepdims=True)
    out_ref[...] = (t2 - mu2) / jnp.sqrt(var2 + 1e-5) * g2_ref[...] + b2_ref[...]


def _post_call(x, wv_parts, z_parts, w_o, b_o, ln1_g, ln1_b, w1, bb1, w2,
               bb2, ln2_g, ln2_b):
    br = 1000
    grid = (N // br,)
    full = lambda shape: pl.BlockSpec(shape, lambda i: tuple(0 for _ in shape))
    return pl.pallas_call(
        _post_body,
        grid=grid,
        in_specs=[
            pl.BlockSpec((br, NDIM), lambda i: (i, 0)),
            pl.BlockSpec((NC, br, NDIM), lambda i: (0, i, 0)),
            pl.BlockSpec((NC, br, 16), lambda i: (0, i, 0)),
            full((NDIM, NDIM)),
            full((1, NDIM)),
            full((1, NDIM)),
            full((1, NDIM)),
            full((NDIM, 4 * NDIM)),
            full((1, 4 * NDIM)),
            full((4 * NDIM, NDIM)),
            full((1, NDIM)),
            full((1, NDIM)),
            full((1, NDIM)),
        ],
        out_specs=pl.BlockSpec((br, NDIM), lambda i: (i, 0)),
        out_shape=jax.ShapeDtypeStruct((N, NDIM), jnp.float32),
    )(x, wv_parts, z_parts, w_o, b_o, ln1_g, ln1_b, w1, bb1, w2, bb2,
      ln2_g, ln2_b)


def kernel(x, lgx, edge_index, W_q, b_q, W_k, W_v, W_o, b_o, ln1_g, ln1_b,
           W1, b1, W2, b2, ln2_g, ln2_b):
    wqkv = jnp.concatenate([W_q, W_k, W_v], axis=1)
    bqkv = jnp.concatenate(
        [b_q, jnp.zeros((2 * NDIM,), jnp.float32)])[None, :]
    q, k, v = _qkv_call(x, wqkv, bqkv)

    src = edge_index[0]
    dst = edge_index[1]
    dstg = dst // 8
    wv_parts, z_packed = _edge_kernel(q, k, v, lgx, src, dst, dstg)
    z_parts = z_packed[:, :NG, :].reshape(NC, N, 16)

    out = _post_call(x, wv_parts, z_parts, W_o, b_o[None, :],
                     ln1_g[None, :], ln1_b[None, :], W1, b1[None, :],
                     W2, b2[None, :], ln2_g[None, :], ln2_b[None, :])
    return (out, lgx)


# speedup vs baseline: 25.2121x; 1.4549x over previous
import functools

import jax
import jax.numpy as jnp
from jax import lax
from jax.experimental import pallas as pl
from jax.experimental.pallas import tpu as pltpu
from jax.experimental.pallas import tpu_sc as plsc

N = 10000
E = 320000
NDIM = 128
EDIM = 16
H = 8
DK = 16

NC = 2
NS = 16
NW = NC * NS
EPW = E // NW
C = 40
NCHUNK = EPW // C
RCH = 16
NRCH = N // RCH
NBCH = (NRCH + NS - 1) // NS
NG = N // 8
NGCH = (NG + RCH - 1) // RCH
NGP = NGCH * RCH
NBG = (NGCH + NS - 1) // NS


def _qkv_body(x_ref, w_ref, b_ref, q_ref, k_ref, v_ref):
    acc = jnp.dot(x_ref[...], w_ref[...], preferred_element_type=jnp.float32)
    acc = acc + b_ref[...]
    q_ref[...] = acc[:, 0:NDIM]
    k_ref[...] = acc[:, NDIM:2 * NDIM]
    v_ref[...] = acc[:, 2 * NDIM:3 * NDIM]


def _qkv_call(x, wqkv, bqkv):
    br = 1000
    grid = (N // br,)
    return pl.pallas_call(
        _qkv_body,
        grid=grid,
        in_specs=[
            pl.BlockSpec((br, NDIM), lambda i: (i, 0)),
            pl.BlockSpec((NDIM, 3 * NDIM), lambda i: (0, 0)),
            pl.BlockSpec((1, 3 * NDIM), lambda i: (0, 0)),
        ],
        out_specs=[
            pl.BlockSpec((br, NDIM), lambda i: (i, 0)),
            pl.BlockSpec((br, NDIM), lambda i: (i, 0)),
            pl.BlockSpec((br, NDIM), lambda i: (i, 0)),
        ],
        out_shape=[
            jax.ShapeDtypeStruct((N, NDIM), jnp.float32),
            jax.ShapeDtypeStruct((N, NDIM), jnp.float32),
            jax.ShapeDtypeStruct((N, NDIM), jnp.float32),
        ],
    )(x, wqkv, bqkv)


def _dyng(v, idx):
    return lax.gather(
        v, idx[:, None],
        dimension_numbers=lax.GatherDimensionNumbers(
            offset_dims=(), collapsed_slice_dims=(0,), start_index_map=(0,)),
        slice_sizes=(1,), mode=lax.GatherScatterMode.PROMISE_IN_BOUNDS)


_SC_MESH = plsc.VectorSubcoreMesh(core_axis_name="c", subcore_axis_name="s")


@functools.partial(
    pl.kernel,
    mesh=_SC_MESH,
    out_type=(
        jax.ShapeDtypeStruct((NC, N, NDIM), jnp.float32),
        jax.ShapeDtypeStruct((NC, NGP, NDIM), jnp.float32),
    ),
    scratch_types=[
        pltpu.VMEM((C,), jnp.int32),
        pltpu.VMEM((C + 16,), jnp.int32),
        pltpu.VMEM((C,), jnp.int32),
        pltpu.VMEM((C, NDIM), jnp.float32),
        pltpu.VMEM((C, NDIM), jnp.float32),
        pltpu.VMEM((C, NDIM), jnp.float32),
        pltpu.VMEM((C, EDIM), jnp.float32),
        pltpu.VMEM((C, NDIM), jnp.float32),
        pltpu.VMEM((C, NDIM), jnp.float32),
        pltpu.VMEM((C,), jnp.int32),
        pltpu.VMEM((16,), jnp.int32),
        pltpu.VMEM_SHARED((N, NDIM), jnp.float32),
        pltpu.VMEM_SHARED((NGP, NDIM), jnp.float32),
        pltpu.SemaphoreType.DMA,
        pltpu.SemaphoreType.DMA,
        pltpu.SemaphoreType.DMA,
        pltpu.SemaphoreType.DMA,
    ],
)
def _edge_kernel(q_hbm, k_hbm, v_hbm, lgx_hbm, src_hbm, dst_hbm, dstg_hbm,
                 wv_out, z_out,
                 src_v, dst_v, dsti_v, k_v, q_v, v_v, e_v, msg_v, zp_v, dstg_v, idx_v,
                 wv_sh, z_sh, sem0, sem1, sem2, sem3):
    cid = lax.axis_index("c")
    sid = lax.axis_index("s")
    wid = sid * NC + cid

    zv = jnp.zeros((16,), jnp.float32)

    def _zrow(r, carry):
        for j in range(NDIM // 16):
            msg_v[r, pl.ds(j * 16, 16)] = zv
            zp_v[r, pl.ds(j * 16, 16)] = zv
        return carry
    lax.fori_loop(0, C, _zrow, 0)

    lane = lax.iota(jnp.int32, 16)

    def _zchunk(i, carry):
        off = jnp.minimum(sid + i * NS, NRCH - 1) * RCH
        idx_v[pl.ds(0, 16)] = off + lane
        pltpu.async_copy(msg_v.at[pl.ds(0, RCH)],
                         wv_sh.at[idx_v], sem0).wait()
        return carry
    lax.fori_loop(0, NBCH, _zchunk, 0)

    def _zgchunk(i, carry):
        off = jnp.minimum(sid + i * NS, NGCH - 1) * RCH
        idx_v[pl.ds(0, 16)] = off + lane
        pltpu.async_copy(zp_v.at[pl.ds(0, RCH)],
                         z_sh.at[idx_v], sem1).wait()
        return carry
    lax.fori_loop(0, NBG, _zgchunk, 0)
    plsc.subcore_barrier()

    def _chunk(ci, carry):
        gbase = wid * EPW + ci * C
        cp0 = pltpu.async_copy(src_hbm.at[pl.ds(gbase, C)], src_v, sem0)
        cp1 = pltpu.async_copy(dst_hbm.at[pl.ds(gbase, C)],
                               dst_v.at[pl.ds(0, C)], sem1)
        cp2 = pltpu.async_copy(dst_hbm.at[pl.ds(gbase, C)], dsti_v, sem2)
        cp3 = pltpu.async_copy(dstg_hbm.at[pl.ds(gbase, C)], dstg_v, sem3)
        cp0.wait(); cp1.wait(); cp2.wait(); cp3.wait()
        cp0 = pltpu.async_copy(k_hbm.at[src_v], k_v, sem0)
        cp1 = pltpu.async_copy(v_hbm.at[src_v], v_v, sem1)
        cp2 = pltpu.async_copy(q_hbm.at[dsti_v], q_v, sem2)
        cp3 = pltpu.async_copy(lgx_hbm.at[pl.ds(gbase, C)], e_v, sem3)
        cp0.wait(); cp1.wait(); cp2.wait(); cp3.wait()

        def _edge(ei, ecarry):
            ef = e_v[ei, pl.ds(0, EDIM)]
            zvec = jnp.zeros((16,), jnp.float32)
            for h in range(H):
                kk = k_v[ei, pl.ds(h * DK, DK)]
                qq = q_v[ei, pl.ds(h * DK, DK)]
                p = (kk + ef) * qq
                for st in (8, 4, 2, 1):
                    p = p + _dyng(p, lane ^ st)
                zvec = jnp.where(lane == h, p, zvec)
            ez = jnp.exp(jnp.minimum(jnp.maximum(zvec * 0.25, -5.0), 5.0))
            for h in range(H):
                ph = _dyng(ez, jnp.full((16,), h, jnp.int32))
                vv = v_v[ei, pl.ds(h * DK, DK)]
                msg_v[ei, pl.ds(h * DK, DK)] = (vv + ef) * ph
            base = (ei // 16) * 16
            dvec = dst_v[pl.ds(base, 16)] & 7
            slot = _dyng(dvec, jnp.full((16,), ei - base, jnp.int32))
            slotf = slot.astype(jnp.float32)
            for j in range(8):
                d = slotf - float(j)
                ind = jnp.maximum(1.0 - d * d, 0.0)
                zp_v[ei, pl.ds(j * 16, 16)] = ez * ind
            return ecarry
        lax.fori_loop(0, C, _edge, 0)

        pltpu.async_copy(msg_v, wv_sh.at[dsti_v], sem0, add=True).wait()
        pltpu.async_copy(zp_v, z_sh.at[dstg_v], sem1, add=True).wait()
        return carry
    lax.fori_loop(0, NCHUNK, _chunk, 0)

    plsc.subcore_barrier()

    def _wchunk(i, carry):
        off = jnp.minimum(sid + i * NS, NRCH - 1) * RCH
        idx_v[pl.ds(0, 16)] = off + lane
        pltpu.async_copy(wv_sh.at[idx_v],
                         msg_v.at[pl.ds(0, RCH)], sem0).wait()
        pltpu.async_copy(msg_v.at[pl.ds(0, RCH)],
                         wv_out.at[cid, pl.ds(off, RCH)], sem1).wait()
        return carry
    lax.fori_loop(0, NBCH, _wchunk, 0)

    def _wgchunk(i, carry):
        off = jnp.minimum(sid + i * NS, NGCH - 1) * RCH
        idx_v[pl.ds(0, 16)] = off + lane
        pltpu.async_copy(z_sh.at[idx_v],
                         zp_v.at[pl.ds(0, RCH)], sem2).wait()
        pltpu.async_copy(zp_v.at[pl.ds(0, RCH)],
                         z_out.at[cid, pl.ds(off, RCH)], sem3).wait()
        return carry
    lax.fori_loop(0, NBG, _wgchunk, 0)


def _post_body(x_ref, wv_ref, z_ref, wo_ref, bo_ref, g1_ref, b1_ref,
               w1_ref, bb1_ref, w2_ref, bb2_ref, g2_ref, b2_ref, out_ref):
    wv = wv_ref[0] + wv_ref[1]
    z = z_ref[0] + z_ref[1]
    rows = lax.broadcasted_iota(jnp.int32, (16, NDIM), 0)
    cols = lax.broadcasted_iota(jnp.int32, (16, NDIM), 1)
    sel = (rows == cols // DK).astype(jnp.float32)
    zfull = jnp.dot(z, sel, preferred_element_type=jnp.float32)
    o = wv / zfull

    t = x_ref[...] + jnp.dot(o, wo_ref[...],
                             preferred_element_type=jnp.float32) + bo_ref[...]
    mu = jnp.mean(t, axis=-1, keepdims=True)
    var = jnp.mean((t - mu) ** 2, axis=-1, keepdims=True)
    t1 = (t - mu) / jnp.sqrt(var + 1e-5) * g1_ref[...] + b1_ref[...]

    hdn = jnp.maximum(
        jnp.dot(t1, w1_ref[...], preferred_element_type=jnp.float32)
        + bb1_ref[...], 0.0)
    ff = jnp.dot(hdn, w2_ref[...],
                 preferred_element_type=jnp.float32) + bb2_ref[...]

    t2 = t1 + ff
    mu2 = jnp.mean(t2, axis=-1, keepdims=True)
    var2 = jnp.mean((t2 - mu2) ** 2, axis=-1, keepdims=True)
    out_ref[...] = (t2 - mu2) / jnp.sqrt(var2 + 1e-5) * g2_ref[...] + b2_ref[...]


def _post_call(x, wv_parts, z_parts, w_o, b_o, ln1_g, ln1_b, w1, bb1, w2,
               bb2, ln2_g, ln2_b):
    br = 1000
    grid = (N // br,)
    full = lambda shape: pl.BlockSpec(shape, lambda i: tuple(0 for _ in shape))
    return pl.pallas_call(
        _post_body,
        grid=grid,
        in_specs=[
            pl.BlockSpec((br, NDIM), lambda i: (i, 0)),
            pl.BlockSpec((NC, br, NDIM), lambda i: (0, i, 0)),
            pl.BlockSpec((NC, br, 16), lambda i: (0, i, 0)),
            full((NDIM, NDIM)),
            full((1, NDIM)),
            full((1, NDIM)),
            full((1, NDIM)),
            full((NDIM, 4 * NDIM)),
            full((1, 4 * NDIM)),
            full((4 * NDIM, NDIM)),
            full((1, NDIM)),
            full((1, NDIM)),
            full((1, NDIM)),
        ],
        out_specs=pl.BlockSpec((br, NDIM), lambda i: (i, 0)),
        out_shape=jax.ShapeDtypeStruct((N, NDIM), jnp.float32),
    )(x, wv_parts, z_parts, w_o, b_o, ln1_g, ln1_b, w1, bb1, w2, bb2,
      ln2_g, ln2_b)


def kernel(x, lgx, edge_index, W_q, b_q, W_k, W_v, W_o, b_o, ln1_g, ln1_b,
           W1, b1, W2, b2, ln2_g, ln2_b):
    wqkv = jnp.concatenate([W_q, W_k, W_v], axis=1)
    bqkv = jnp.concatenate(
        [b_q, jnp.zeros((2 * NDIM,), jnp.float32)])[None, :]
    q, k, v = _qkv_call(x, wqkv, bqkv)

    src = edge_index[0]
    dst = edge_index[1]
    dstg = dst // 8
    wv_parts, z_packed = _edge_kernel(q, k, v, lgx, src, dst, dstg)
    z_parts = z_packed[:, :NG, :].reshape(NC, N, 16)

    out = _post_call(x, wv_parts, z_parts, W_o, b_o[None, :],
                     ln1_g[None, :], ln1_b[None, :], W1, b1[None, :],
                     W2, b2[None, :], ln2_g[None, :], ln2_b[None, :])
    return (out, lgx)
